# Initial kernel scaffold; baseline (speedup 1.0000x reference)
import jax
import jax.numpy as jnp
from jax.experimental import pallas as pl


def kernel(x, f_ij, idx_i, idx_j, rcut_ij, W_in, b_in, W_f1, b_f1, W_f2, b_f2, W_o1, b_o1, W_o2, b_o2):
    raise NotImplementedError("write your pallas kernel here")


# speedup vs baseline: 1.9508x; 1.9508x over previous
import functools

import jax
import jax.numpy as jnp
from jax import lax
from jax.experimental import pallas as pl
from jax.experimental.pallas import tpu as pltpu
from jax.experimental.pallas import tpu_sc as plsc

N_FEAT = 128
N_NODES = 10000
N_EDGES = 320000
N_RBF = 20

_LOG2 = 0.6931471805599453


def _ssp(v):
    return jnp.maximum(v, 0.0) + jnp.log1p(jnp.exp(-jnp.abs(v))) - _LOG2


def _h_body(x_ref, w_ref, b_ref, o_ref):
    o_ref[...] = (
        jnp.dot(x_ref[...], w_ref[...], preferred_element_type=jnp.float32)
        + b_ref[...]
    )


def _wij_body(f_ref, rc_ref, w1_ref, b1_ref, w2_ref, b2_ref, o_ref):
    w = jnp.dot(f_ref[...], w1_ref[...], preferred_element_type=jnp.float32)
    w = _ssp(w + b1_ref[...])
    w = jnp.dot(w, w2_ref[...], preferred_element_type=jnp.float32) + b2_ref[...]
    o_ref[...] = w * rc_ref[...]


def _out_body(p_ref, w1_ref, b1_ref, w2_ref, b2_ref, o_ref):
    agg = p_ref[0] + p_ref[1]
    o = jnp.dot(agg, w1_ref[...], preferred_element_type=jnp.float32)
    o = _ssp(o + b1_ref[...])
    o_ref[...] = (
        jnp.dot(o, w2_ref[...], preferred_element_type=jnp.float32) + b2_ref[...]
    )


_NTILES = 32
_EPT = N_EDGES // _NTILES
_C = 80
_NCHUNK = _EPT // _C
_SLAB = 80
_NSLAB = N_NODES // _SLAB


def _sc_body(h_hbm, wij_hbm, idxj_hbm, idxi_hbm, out_hbm,
             idxj_v, idxi_v, rows_v, wij_v, zbuf_v, acc_sh, sem):
    core = lax.axis_index("c")
    sub = lax.axis_index("s")
    tile_base = pl.multiple_of((sub * 2 + core) * _EPT, _EPT)

    def zrow(r, _):
        for cb in range(N_FEAT // 16):
            zbuf_v[r, pl.ds(cb * 16, 16)] = jnp.zeros((16,), jnp.float32)
        return 0
    lax.fori_loop(0, _SLAB, zrow, 0)
    for t in range((_NSLAB + 15) // 16):
        sl = sub + 16 * t

        @pl.when(sl < _NSLAB)
        def _():
            off = pl.multiple_of(sl * _SLAB, _SLAB)
            pltpu.sync_copy(zbuf_v, acc_sh.at[pl.ds(off, _SLAB)])
    plsc.subcore_barrier()

    def chunk(k, _):
        base = pl.multiple_of(tile_base + k * _C, _C)
        pltpu.sync_copy(idxj_hbm.at[pl.ds(base, _C)], idxj_v)
        pltpu.sync_copy(idxi_hbm.at[pl.ds(base, _C)], idxi_v)
        pltpu.sync_copy(wij_hbm.at[pl.ds(base, _C)], wij_v)
        pltpu.async_copy(h_hbm.at[idxj_v], rows_v, sem).wait()

        def mul(r, _):
            for cb in range(N_FEAT // 16):
                sl = pl.ds(cb * 16, 16)
                rows_v[r, sl] = rows_v[r, sl] * wij_v[r, sl]
            return 0
        lax.fori_loop(0, _C, mul, 0)

        pltpu.sync_copy(rows_v, acc_sh.at[idxi_v], add=True)
        return 0
    lax.fori_loop(0, _NCHUNK, chunk, 0)

    plsc.subcore_barrier()
    for t in range((_NSLAB + 15) // 16):
        sl = sub + 16 * t

        @pl.when(sl < _NSLAB)
        def _():
            off = pl.multiple_of(sl * _SLAB, _SLAB)
            pltpu.sync_copy(acc_sh.at[pl.ds(off, _SLAB)],
                            out_hbm.at[core, pl.ds(off, _SLAB)])


@functools.cache
def _sc_edge_stage():
    return pl.kernel(
        _sc_body,
        out_type=jax.ShapeDtypeStruct((2, N_NODES, N_FEAT), jnp.float32),
        mesh=plsc.VectorSubcoreMesh(core_axis_name="c", subcore_axis_name="s"),
        scratch_types=[
            pltpu.VMEM((_C,), jnp.int32),
            pltpu.VMEM((_C,), jnp.int32),
            pltpu.VMEM((_C, N_FEAT), jnp.float32),
            pltpu.VMEM((_C, N_FEAT), jnp.float32),
            pltpu.VMEM((_SLAB, N_FEAT), jnp.float32),
            pltpu.VMEM_SHARED((N_NODES, N_FEAT), jnp.float32),
            pltpu.SemaphoreType.DMA,
        ],
    )


def kernel(x, f_ij, idx_i, idx_j, rcut_ij,
           W_in, b_in, W_f1, b_f1, W_f2, b_f2,
           W_o1, b_o1, W_o2, b_o2):
    batch, atoms, feat = x.shape
    x2 = x.reshape(batch * atoms, feat)

    mb = 2000
    h = pl.pallas_call(
        _h_body,
        grid=(N_NODES // mb,),
        in_specs=[
            pl.BlockSpec((mb, feat), lambda i: (i, 0)),
            pl.BlockSpec((feat, N_FEAT), lambda i: (0, 0)),
            pl.BlockSpec((1, N_FEAT), lambda i: (0, 0)),
        ],
        out_specs=pl.BlockSpec((mb, N_FEAT), lambda i: (i, 0)),
        out_shape=jax.ShapeDtypeStruct((N_NODES, N_FEAT), jnp.float32),
    )(x2, W_in, b_in.reshape(1, N_FEAT))

    eb = 3200
    wij = pl.pallas_call(
        _wij_body,
        grid=(N_EDGES // eb,),
        in_specs=[
            pl.BlockSpec((eb, N_RBF), lambda i: (i, 0)),
            pl.BlockSpec((eb, 1), lambda i: (i, 0)),
            pl.BlockSpec((N_RBF, N_FEAT), lambda i: (0, 0)),
            pl.BlockSpec((1, N_FEAT), lambda i: (0, 0)),
            pl.BlockSpec((N_FEAT, N_FEAT), lambda i: (0, 0)),
            pl.BlockSpec((1, N_FEAT), lambda i: (0, 0)),
        ],
        out_specs=pl.BlockSpec((eb, N_FEAT), lambda i: (i, 0)),
        out_shape=jax.ShapeDtypeStruct((N_EDGES, N_FEAT), jnp.float32),
    )(f_ij, rcut_ij.reshape(N_EDGES, 1), W_f1, b_f1.reshape(1, N_FEAT),
      W_f2, b_f2.reshape(1, N_FEAT))

    partials = _sc_edge_stage()(
        h, wij, idx_j.astype(jnp.int32), idx_i.astype(jnp.int32))

    ob = 2000
    out = pl.pallas_call(
        _out_body,
        grid=(N_NODES // ob,),
        in_specs=[
            pl.BlockSpec((2, ob, N_FEAT), lambda i: (0, i, 0)),
            pl.BlockSpec((N_FEAT, N_FEAT), lambda i: (0, 0)),
            pl.BlockSpec((1, N_FEAT), lambda i: (0, 0)),
            pl.BlockSpec((N_FEAT, N_FEAT), lambda i: (0, 0)),
            pl.BlockSpec((1, N_FEAT), lambda i: (0, 0)),
        ],
        out_specs=pl.BlockSpec((ob, N_FEAT), lambda i: (i, 0)),
        out_shape=jax.ShapeDtypeStruct((N_NODES, N_FEAT), jnp.float32),
    )(partials, W_o1, b_o1.reshape(1, N_FEAT), W_o2, b_o2.reshape(1, N_FEAT))

    return out.reshape(batch, atoms, N_FEAT)


# speedup vs baseline: 2.4918x; 1.2773x over previous
import functools

import jax
import jax.numpy as jnp
from jax import lax
from jax.experimental import pallas as pl
from jax.experimental.pallas import tpu as pltpu
from jax.experimental.pallas import tpu_sc as plsc

N_FEAT = 128
N_NODES = 10000
N_EDGES = 320000
N_RBF = 20

_LOG2 = 0.6931471805599453


def _ssp(v):
    return jnp.maximum(v, 0.0) + jnp.log1p(jnp.exp(-jnp.abs(v))) - _LOG2


def _h_body(x_ref, w_ref, b_ref, o_ref):
    o_ref[...] = (
        jnp.dot(x_ref[...], w_ref[...], preferred_element_type=jnp.float32)
        + b_ref[...]
    )


def _wij_body(f_ref, rc_ref, w1_ref, b1_ref, w2_ref, b2_ref, o_ref):
    w = jnp.dot(f_ref[...], w1_ref[...], preferred_element_type=jnp.float32)
    w = _ssp(w + b1_ref[...])
    w = jnp.dot(w, w2_ref[...], preferred_element_type=jnp.float32) + b2_ref[...]
    o_ref[...] = w * rc_ref[...]


def _out_body(p_ref, w1_ref, b1_ref, w2_ref, b2_ref, o_ref):
    agg = p_ref[0] + p_ref[1]
    o = jnp.dot(agg, w1_ref[...], preferred_element_type=jnp.float32)
    o = _ssp(o + b1_ref[...])
    o_ref[...] = (
        jnp.dot(o, w2_ref[...], preferred_element_type=jnp.float32) + b2_ref[...]
    )


_NTILES = 32
_EPT = N_EDGES // _NTILES
_C = 40
_NCHUNK = _EPT // _C
_SLAB = 40
_NSLAB = N_NODES // _SLAB


def _sc_body(h_hbm, wij_hbm, idxj_hbm, idxi_hbm, out_hbm,
             idxj0_v, idxj1_v, idxi0_v, idxi1_v,
             rows0_v, rows1_v, wij0_v, wij1_v, acc_sh,
             jsem0, jsem1, msem0, msem1,
             gsem0, gsem1, wsem0, wsem1, ssem0, ssem1):
    core = lax.axis_index("c")
    sub = lax.axis_index("s")
    tile_base = pl.multiple_of((sub * 2 + core) * _EPT, _EPT)

    def zrow(r, _):
        for cb in range(N_FEAT // 16):
            rows0_v[r, pl.ds(cb * 16, 16)] = jnp.zeros((16,), jnp.float32)
        return 0
    lax.fori_loop(0, _SLAB, zrow, 0)
    for t in range((_NSLAB + 15) // 16):
        sl = sub + 16 * t

        @pl.when(sl < _NSLAB)
        def _():
            off = pl.multiple_of(sl * _SLAB, _SLAB)
            pltpu.sync_copy(rows0_v, acc_sh.at[pl.ds(off, _SLAB)])
    plsc.subcore_barrier()

    idxj = (idxj0_v, idxj1_v)
    idxi = (idxi0_v, idxi1_v)
    rows = (rows0_v, rows1_v)
    wijb = (wij0_v, wij1_v)
    jsem = (jsem0, jsem1)
    msem = (msem0, msem1)
    gsem = (gsem0, gsem1)
    wsem = (wsem0, wsem1)
    ssem = (ssem0, ssem1)

    def mul(rv, wv):
        def body(r, _):
            for cb in range(N_FEAT // 16):
                sl = pl.ds(cb * 16, 16)
                rv[r, sl] = rv[r, sl] * wv[r, sl]
            return 0
        lax.fori_loop(0, _C, body, 0)

    def pair(g, _):
        dj, di, dw = [], [], []
        for b in range(2):
            base = pl.multiple_of(tile_base + (g * 2 + b) * _C, _C)
            dj.append(pltpu.async_copy(idxj_hbm.at[pl.ds(base, _C)], idxj[b],
                                       jsem[b]))
            di.append(pltpu.async_copy(idxi_hbm.at[pl.ds(base, _C)], idxi[b],
                                       msem[b]))
            dw.append(pltpu.async_copy(wij_hbm.at[pl.ds(base, _C)], wijb[b],
                                       wsem[b]))
        dg = []
        for b in range(2):
            dj[b].wait()
            dg.append(pltpu.async_copy(h_hbm.at[idxj[b]], rows[b], gsem[b]))
        dsc = []
        for b in range(2):
            dg[b].wait()
            dw[b].wait()
            mul(rows[b], wijb[b])
            di[b].wait()
            dsc.append(pltpu.async_copy(rows[b], acc_sh.at[idxi[b]],
                                        ssem[b], add=True))
        for b in range(2):
            dsc[b].wait()
        return 0
    lax.fori_loop(0, _NCHUNK // 2, pair, 0)

    plsc.subcore_barrier()
    for t in range((_NSLAB + 15) // 16):
        sl = sub + 16 * t

        @pl.when(sl < _NSLAB)
        def _():
            off = pl.multiple_of(sl * _SLAB, _SLAB)
            pltpu.sync_copy(acc_sh.at[pl.ds(off, _SLAB)],
                            out_hbm.at[core, pl.ds(off, _SLAB)])


@functools.cache
def _sc_edge_stage():
    return pl.kernel(
        _sc_body,
        out_type=jax.ShapeDtypeStruct((2, N_NODES, N_FEAT), jnp.float32),
        mesh=plsc.VectorSubcoreMesh(core_axis_name="c", subcore_axis_name="s"),
        scratch_types=[
            pltpu.VMEM((_C,), jnp.int32),
            pltpu.VMEM((_C,), jnp.int32),
            pltpu.VMEM((_C,), jnp.int32),
            pltpu.VMEM((_C,), jnp.int32),
            pltpu.VMEM((_C, N_FEAT), jnp.float32),
            pltpu.VMEM((_C, N_FEAT), jnp.float32),
            pltpu.VMEM((_C, N_FEAT), jnp.float32),
            pltpu.VMEM((_C, N_FEAT), jnp.float32),
            pltpu.VMEM_SHARED((N_NODES, N_FEAT), jnp.float32),
        ] + [pltpu.SemaphoreType.DMA] * 10,
    )


def kernel(x, f_ij, idx_i, idx_j, rcut_ij,
           W_in, b_in, W_f1, b_f1, W_f2, b_f2,
           W_o1, b_o1, W_o2, b_o2):
    batch, atoms, feat = x.shape
    x2 = x.reshape(batch * atoms, feat)

    mb = 2000
    h = pl.pallas_call(
        _h_body,
        grid=(N_NODES // mb,),
        in_specs=[
            pl.BlockSpec((mb, feat), lambda i: (i, 0)),
            pl.BlockSpec((feat, N_FEAT), lambda i: (0, 0)),
            pl.BlockSpec((1, N_FEAT), lambda i: (0, 0)),
        ],
        out_specs=pl.BlockSpec((mb, N_FEAT), lambda i: (i, 0)),
        out_shape=jax.ShapeDtypeStruct((N_NODES, N_FEAT), jnp.float32),
    )(x2, W_in, b_in.reshape(1, N_FEAT))

    eb = 3200
    wij = pl.pallas_call(
        _wij_body,
        grid=(N_EDGES // eb,),
        in_specs=[
            pl.BlockSpec((eb, N_RBF), lambda i: (i, 0)),
            pl.BlockSpec((eb, 1), lambda i: (i, 0)),
            pl.BlockSpec((N_RBF, N_FEAT), lambda i: (0, 0)),
            pl.BlockSpec((1, N_FEAT), lambda i: (0, 0)),
            pl.BlockSpec((N_FEAT, N_FEAT), lambda i: (0, 0)),
            pl.BlockSpec((1, N_FEAT), lambda i: (0, 0)),
        ],
        out_specs=pl.BlockSpec((eb, N_FEAT), lambda i: (i, 0)),
        out_shape=jax.ShapeDtypeStruct((N_EDGES, N_FEAT), jnp.float32),
    )(f_ij, rcut_ij.reshape(N_EDGES, 1), W_f1, b_f1.reshape(1, N_FEAT),
      W_f2, b_f2.reshape(1, N_FEAT))

    partials = _sc_edge_stage()(
        h, wij, idx_j.astype(jnp.int32), idx_i.astype(jnp.int32))

    ob = 2000
    out = pl.pallas_call(
        _out_body,
        grid=(N_NODES // ob,),
        in_specs=[
            pl.BlockSpec((2, ob, N_FEAT), lambda i: (0, i, 0)),
            pl.BlockSpec((N_FEAT, N_FEAT), lambda i: (0, 0)),
            pl.BlockSpec((1, N_FEAT), lambda i: (0, 0)),
            pl.BlockSpec((N_FEAT, N_FEAT), lambda i: (0, 0)),
            pl.BlockSpec((1, N_FEAT), lambda i: (0, 0)),
        ],
        out_specs=pl.BlockSpec((ob, N_FEAT), lambda i: (i, 0)),
        out_shape=jax.ShapeDtypeStruct((N_NODES, N_FEAT), jnp.float32),
    )(partials, W_o1, b_o1.reshape(1, N_FEAT), W_o2, b_o2.reshape(1, N_FEAT))

    return out.reshape(batch, atoms, N_FEAT)
